# Initial kernel scaffold; baseline (speedup 1.0000x reference)
import jax
import jax.numpy as jnp
from jax.experimental import pallas as pl


def kernel(tokens, table):
    raise NotImplementedError("write your pallas kernel here")


# speedup vs baseline: 1.3316x; 1.3316x over previous
import math

import jax
import jax.numpy as jnp
import numpy as np
from jax import lax
from jax.experimental import pallas as pl
from jax.experimental.pallas import tpu as pltpu
from jax.experimental.pallas import tpu_sc as plsc

D_MODEL = 128
MAX_LEN = 1500
BATCH = 256
SEQ = 1024

NC, NS, LANES = 2, 16, 16
NW = NC * NS
TOK_PER_W = BATCH * SEQ // NW
QUARTERS = 4
Q = SEQ // QUARTERS
SEQ_PER_W = TOK_PER_W // SEQ


def _make_pe_np(max_len, d_model):
    position = np.arange(0, max_len, dtype=np.float32)[:, None]
    div_term = np.exp(
        np.arange(0, d_model, 2).astype(np.float32) * -(math.log(10000.0) / d_model)
    )
    pe = np.zeros((max_len, d_model), dtype=np.float32)
    pe[:, 0::2] = np.sin(position * div_term)
    pe[:, 1::2] = np.cos(position * div_term)
    return pe


_PE = jnp.asarray(_make_pe_np(MAX_LEN, D_MODEL)[:SEQ])


def _sc_embed(tokens_flat, table, pe):
    mesh = plsc.VectorSubcoreMesh(
        core_axis_name="c", subcore_axis_name="s", num_cores=NC, num_subcores=NS
    )

    def body(tok_hbm, table_hbm, pe_hbm, out_hbm, idx_v, rows_v, pe_v, sem):
        wid = lax.axis_index("s") * NC + lax.axis_index("c")
        base = wid * TOK_PER_W
        for q in range(QUARTERS):
            pltpu.sync_copy(pe_hbm.at[pl.ds(q * Q, Q)], pe_v)

            def seq_body(s, carry, q=q):
                g = base + s * SEQ + q * Q
                pltpu.sync_copy(tok_hbm.at[pl.ds(g, Q)], idx_v)
                pltpu.async_copy(table_hbm.at[idx_v], rows_v, sem).wait()

                def addrow(r, c2):
                    for j in range(D_MODEL // LANES):
                        sl = pl.ds(j * LANES, LANES)
                        plsc.addupdate(rows_v.at[r, sl], pe_v[r, sl])
                    return c2

                lax.fori_loop(0, Q, addrow, 0)
                pltpu.sync_copy(rows_v, out_hbm.at[pl.ds(g, Q)])
                return carry

            lax.fori_loop(0, SEQ_PER_W, seq_body, 0)

    run = pl.kernel(
        body,
        out_type=jax.ShapeDtypeStruct((BATCH * SEQ, D_MODEL), jnp.float32),
        mesh=mesh,
        scratch_types=[
            pltpu.VMEM((Q,), jnp.int32),
            pltpu.VMEM((Q, D_MODEL), jnp.float32),
            pltpu.VMEM((Q, D_MODEL), jnp.float32),
            pltpu.SemaphoreType.DMA,
        ],
    )
    return run(tokens_flat, table, pe)


def kernel(tokens, table):
    tokens_flat = tokens.reshape(-1).astype(jnp.int32)
    out = _sc_embed(tokens_flat, table, _PE)
    return out.reshape(BATCH, SEQ, D_MODEL)
